# baseline (speedup 1.0000x reference)
import jax
import jax.numpy as jnp
from jax import lax
from jax.experimental import pallas as pl
from jax.experimental.pallas import tpu as pltpu
from jax.experimental.pallas import tpu_sc as plsc

B = 16384
NC = 2
NS = 16
NW = NC * NS
RPW = B // NW
G = RPW // 16


def _sc_body(pbT, psT, pbmT, psmT, tbl_hbm, pb_out, ps_out,
             pbv, psv, pbmv, psmv, tblv, sem):
    wid = lax.axis_index("s") * NC + lax.axis_index("c")
    base = wid * RPW

    cps = [
        pltpu.async_copy(pbT.at[:, pl.ds(base, RPW)], pbv, sem),
        pltpu.async_copy(pbmT.at[:, pl.ds(base, RPW)], pbmv, sem),
        pltpu.async_copy(psT.at[:, pl.ds(base, RPW)], psv, sem),
        pltpu.async_copy(psmT.at[:, pl.ds(base, RPW)], psmv, sem),
        pltpu.async_copy(tbl_hbm, tblv, sem),
    ]
    for c in cps:
        c.wait()

    def const(k):
        return tblv[pl.ds(16 * k, 16)]

    d_pb = [const(k) for k in range(9)]
    d_def = [const(9 + k) for k in range(2)]
    d_nw = [const(11 + k) for k in range(2)]
    d_w = [const(13 + k) for k in range(2)]
    d_h1tt = [const(15 + k) for k in range(2)]
    d_h1tt_off = [const(17 + k) for k in range(2)]
    d_h1c = [const(19 + k) for k in range(3)]
    d_h1c_on = [const(22 + k) for k in range(3)]
    d_h1c_off = [const(25 + k) for k in range(3)]
    d_h2tt = [const(28 + k) for k in range(2)]
    d_h2tt_off = [const(30 + k) for k in range(2)]
    d_h2c = [const(32 + k) for k in range(2)]
    d_h2c_on = [const(34 + k) for k in range(2)]
    d_h2c_off = [const(36 + k) for k in range(2)]
    d_ps = [const(38 + k) for k in range(6)]

    def group(g, carry):
        off = g * 16

        def ld(buf, j):
            return buf[j, pl.ds(off, 16)]

        mb = [ld(pbmv, j) for j in range(9)]
        b0, b1, b2, b6 = ld(pbv, 0), ld(pbv, 1), ld(pbv, 2), ld(pbv, 6)
        pb1_0 = b0 * mb[0] + (1.0 - mb[0]) * d_pb[0]
        pb1_1 = b1 * mb[1] + (1.0 - mb[1]) * d_pb[1]
        pb1_2 = b2 * mb[2] + (1.0 - mb[2]) * d_pb[2]
        pb1_6 = b6 * mb[6] + (1.0 - mb[6]) * d_pb[6]

        has_nw = mb[0] > 0.5
        hot_nw = pb1_0 > 0.5
        has_w = mb[1] > 0.5
        hot_w = pb1_1 > 0.5
        ht1_known = mb[2] > 0.5
        ht1_hot = pb1_2 > 0.5
        ht2_known = mb[6] > 0.5
        ht2_hot = pb1_6 > 0.5

        pbv[0, pl.ds(off, 16)] = pb1_0
        pbv[1, pl.ds(off, 16)] = pb1_1
        pbv[2, pl.ds(off, 16)] = pb1_2
        pbv[6, pl.ds(off, 16)] = pb1_6
        for i, j in enumerate((3, 4, 5)):
            m = mb[j]
            pb1 = ld(pbv, j) * m + (1.0 - m) * d_pb[j]
            f = jnp.where(
                ht1_known,
                jnp.where(ht1_hot, d_h1c_on[i], d_h1c_off[i]), d_h1c[i])
            pbv[j, pl.ds(off, 16)] = pb1 * m + (1.0 - m) * f
        for i, j in enumerate((7, 8)):
            m = mb[j]
            pb1 = ld(pbv, j) * m + (1.0 - m) * d_pb[j]
            f = jnp.where(
                ht2_known,
                jnp.where(ht2_hot, d_h2c_on[i], d_h2c_off[i]), d_h2c[i])
            pbv[j, pl.ds(off, 16)] = pb1 * m + (1.0 - m) * f

        x0 = jnp.where(has_nw, jnp.where(hot_nw, d_nw[0], d_def[0]), d_def[0])
        x1 = jnp.where(has_nw, jnp.where(hot_nw, d_nw[1], d_def[1]), d_def[1])
        fills = [
            jnp.where(has_w, jnp.where(hot_w, d_w[0], x0), x0),
            jnp.where(has_w, jnp.where(hot_w, d_w[1], x1), x1),
            jnp.where(ht1_known,
                      jnp.where(ht1_hot, d_h1tt[0], d_h1tt_off[0]), d_h1tt[0]),
            jnp.where(ht1_known,
                      jnp.where(ht1_hot, d_h1tt[1], d_h1tt_off[1]), d_h1tt[1]),
            jnp.where(ht2_known,
                      jnp.where(ht2_hot, d_h2tt[0], d_h2tt_off[0]), d_h2tt[0]),
            jnp.where(ht2_known,
                      jnp.where(ht2_hot, d_h2tt[1], d_h2tt_off[1]), d_h2tt[1]),
        ]
        for j in range(6):
            m = ld(psmv, j)
            t = ld(psv, j) * m + (1.0 - m) * fills[j]
            psv[j, pl.ds(off, 16)] = t * m + (1.0 - m) * d_ps[j]
        return carry

    lax.fori_loop(0, G, group, 0)

    cps = [
        pltpu.async_copy(pbv, pb_out.at[:, pl.ds(base, RPW)], sem),
        pltpu.async_copy(psv, ps_out.at[:, pl.ds(base, RPW)], sem),
    ]
    for c in cps:
        c.wait()


def kernel(proc_bool, proc_scalar, proc_bool_mask, proc_scalar_mask,
           p_pb_def, p_def_def, p_def_nw, p_def_w,
           p_ht1_tt_def, p_ht1_tt_off,
           p_ht1_cool_def, p_ht1_cool_on, p_ht1_cool_off,
           p_ht2_tt_def, p_ht2_tt_off,
           p_ht2_cool_def, p_ht2_cool_on, p_ht2_cool_off, p_ps_def):
    prm = jnp.concatenate(
        [p_pb_def, p_def_def, p_def_nw, p_def_w,
         p_ht1_tt_def, p_ht1_tt_off,
         p_ht1_cool_def, p_ht1_cool_on, p_ht1_cool_off,
         p_ht2_tt_def, p_ht2_tt_off,
         p_ht2_cool_def, p_ht2_cool_on, p_ht2_cool_off, p_ps_def])
    tbl = jnp.broadcast_to(prm[:, None], (44, 16)).reshape(-1)

    mesh = plsc.VectorSubcoreMesh(core_axis_name="c", subcore_axis_name="s")
    f32 = jnp.float32
    sck = pl.kernel(
        _sc_body,
        mesh=mesh,
        out_type=[jax.ShapeDtypeStruct((9, B), f32),
                  jax.ShapeDtypeStruct((6, B), f32)],
        scratch_types=[
            pltpu.VMEM((9, RPW), f32),
            pltpu.VMEM((6, RPW), f32),
            pltpu.VMEM((9, RPW), f32),
            pltpu.VMEM((6, RPW), f32),
            pltpu.VMEM((44 * 16,), f32),
            pltpu.SemaphoreType.DMA,
        ],
    )
    pb_out, ps_out = sck(proc_bool.T, proc_scalar.T,
                         proc_bool_mask.T, proc_scalar_mask.T, tbl)
    return (pb_out.T, ps_out.T)
